# baseline (speedup 1.0000x reference)
import functools
import jax
import jax.numpy as jnp
from jax import lax
from jax.experimental import pallas as pl
from jax.experimental.pallas import tpu as pltpu
from jax.experimental.pallas import tpu_sc as plsc

_NC = 2
_NS = 16
_NW = _NC * _NS


def kernel(src, table):
    seq_len, batch = src.shape
    max_len, hidden = table.shape
    flat_cols = batch * hidden

    rows_per_w = seq_len // _NW
    chunk = 32
    n_chunks = rows_per_w // chunk

    mesh = plsc.VectorSubcoreMesh(core_axis_name="c", subcore_axis_name="s")

    @functools.partial(
        pl.kernel,
        mesh=mesh,
        out_type=jax.ShapeDtypeStruct((seq_len, flat_cols), jnp.float32),
        scratch_types=[
            pltpu.VMEM((chunk, hidden), jnp.float32),
            pltpu.VMEM((chunk, hidden), jnp.float32),
            pltpu.SemaphoreType.DMA,
            pltpu.SemaphoreType.DMA,
            pltpu.SemaphoreType.DMA,
        ],
    )
    def k(table_hbm, out_hbm, buf0, buf1, rsem, wsem0, wsem1):
        c = lax.axis_index("c")
        s = lax.axis_index("s")
        wid = s * _NC + c
        base = wid * rows_per_w

        bufs = [buf0, buf1]
        wsems = [wsem0, wsem1]

        def read(j):
            r0 = base + j * chunk
            return pltpu.async_copy(
                table_hbm.at[pl.ds(r0, chunk)], bufs[j % 2], rsem
            )

        def write(j):
            r0 = base + j * chunk
            return [
                pltpu.async_copy(
                    bufs[j % 2],
                    out_hbm.at[pl.ds(r0, chunk), pl.ds(b * hidden, hidden)],
                    wsems[j % 2],
                )
                for b in range(batch)
            ]

        writes = [None] * n_chunks
        pending_read = read(0)
        for j in range(n_chunks):
            pending_read.wait()

            if j == 0:
                @pl.when(wid == 0)
                def _():
                    def zb(i, c2):
                        buf0[0, pl.ds(i * 16, 16)] = jnp.zeros((16,), jnp.float32)
                        return c2
                    lax.fori_loop(0, hidden // 16, zb, 0)

            if j + 1 < n_chunks:
                if j >= 1:
                    for w in writes[j - 1]:
                        w.wait()
                pending_read = read(j + 1)
            writes[j] = write(j)

        for j in (n_chunks - 2, n_chunks - 1):
            for w in writes[j]:
                w.wait()

    out = k(table)
    return out.reshape(seq_len, batch, hidden)


# speedup vs baseline: 3.0003x; 3.0003x over previous
import functools
import jax
import jax.numpy as jnp
from jax import lax
from jax.experimental import pallas as pl
from jax.experimental.pallas import tpu as pltpu
from jax.experimental.pallas import tpu_sc as plsc

_NC = 2
_NS = 16
_NW = _NC * _NS


def kernel(src, table):
    seq_len, batch = src.shape
    max_len, hidden = table.shape
    flat_cols = batch * hidden

    rows_per_w = seq_len // _NW
    chunk = 32
    n_chunks = rows_per_w // chunk

    mesh = plsc.VectorSubcoreMesh(core_axis_name="c", subcore_axis_name="s")

    @functools.partial(
        pl.kernel,
        mesh=mesh,
        out_type=jax.ShapeDtypeStruct((seq_len, batch, hidden), jnp.float32),
        scratch_types=[
            pltpu.VMEM((chunk, hidden), jnp.float32),
            pltpu.VMEM((chunk, hidden), jnp.float32),
            pltpu.SemaphoreType.DMA,
            pltpu.SemaphoreType.DMA,
            pltpu.SemaphoreType.DMA,
        ],
    )
    def k(table_hbm, out_hbm, buf0, buf1, rsem, wsem0, wsem1):
        c = lax.axis_index("c")
        s = lax.axis_index("s")
        wid = s * _NC + c
        base = wid * rows_per_w

        bufs = [buf0, buf1]
        wsems = [wsem0, wsem1]

        def read(j):
            r0 = base + j * chunk
            return pltpu.async_copy(
                table_hbm.at[pl.ds(r0, chunk)], bufs[j % 2], rsem
            )

        def write(j):
            r0 = base + j * chunk
            return [
                pltpu.async_copy(
                    bufs[j % 2],
                    out_hbm.at[pl.ds(r0, chunk), b],
                    wsems[j % 2],
                )
                for b in range(batch)
            ]

        writes = [None] * n_chunks
        pending_read = read(0)
        for j in range(n_chunks):
            pending_read.wait()

            if j == 0:
                @pl.when(wid == 0)
                def _():
                    def zb(i, c2):
                        buf0[0, pl.ds(i * 16, 16)] = jnp.zeros((16,), jnp.float32)
                        return c2
                    lax.fori_loop(0, hidden // 16, zb, 0)

            if j + 1 < n_chunks:
                if j >= 1:
                    for w in writes[j - 1]:
                        w.wait()
                pending_read = read(j + 1)
            writes[j] = write(j)

        for j in (n_chunks - 2, n_chunks - 1):
            for w in writes[j]:
                w.wait()

    return k(table)
